# baseline (speedup 1.0000x reference)
import jax
import jax.numpy as jnp
from jax.experimental import pallas as pl
from jax.experimental.pallas import tpu as pltpu

_BM = 512


def _support_body(x_ref, w_ref, s_ref):
    w = w_ref[...]
    s0 = jnp.dot(x_ref[0], w, preferred_element_type=jnp.float32)
    s1 = jnp.dot(x_ref[1], w, preferred_element_type=jnp.float32)
    s_ref[...] = jnp.concatenate([s0, s1], axis=1)


def _spmm_body(adj_ref, s_ref, out_ref):
    acc = jnp.dot(adj_ref[...], s_ref[...], preferred_element_type=jnp.float32)
    acc = jnp.maximum(acc, 0.0)
    out_ref[0] = acc[:, :16]
    out_ref[1] = acc[:, 16:]


def kernel(input, adj, W):
    bs, n_agents, in_f = input.shape
    out_f = W.shape[1]

    support = pl.pallas_call(
        _support_body,
        out_shape=jax.ShapeDtypeStruct((n_agents, bs * out_f), jnp.float32),
    )(input, W)

    grid = (n_agents // _BM,)
    out = pl.pallas_call(
        _spmm_body,
        grid=grid,
        in_specs=[
            pl.BlockSpec((_BM, n_agents), lambda i: (i, 0)),
            pl.BlockSpec((n_agents, bs * out_f), lambda i: (0, 0)),
        ],
        out_specs=pl.BlockSpec((bs, _BM, out_f), lambda i: (0, i, 0)),
        out_shape=jax.ShapeDtypeStruct((bs, n_agents, out_f), jnp.float32),
        compiler_params=pltpu.CompilerParams(
            dimension_semantics=("parallel",),
        ),
    )(adj, support)

    return out.reshape(bs * n_agents, out_f)
